# baseline (speedup 1.0000x reference)
import functools

import jax
import jax.numpy as jnp
from jax import lax
from jax.experimental import pallas as pl
from jax.experimental.pallas import tpu as pltpu
from jax.experimental.pallas import tpu_sc as plsc

N = 10000
E = 160000
D = 128
K = 16
DW = D // 2

_info = plsc.get_sparse_core_info()
_NC = _info.num_cores
_NS = _info.num_subcores
_NW = _NC * _NS

_CH = 8


def _mesh():
    return plsc.VectorSubcoreMesh(core_axis_name="c", subcore_axis_name="s")


def _tree_sum(vals):
    vals = list(vals)
    while len(vals) > 1:
        nxt = [vals[i] + vals[i + 1] for i in range(0, len(vals) - 1, 2)]
        if len(vals) % 2:
            nxt.append(vals[-1])
        vals = nxt
    return vals[0]


@functools.partial(jax.jit, static_argnames=("rows", "with_p"))
def _sc_gather_fused(tabA, tabB, idxA, idxB, ptable, pidx, *, rows, with_p):
    nch = rows // _CH
    q, rem = divmod(nch, _NW)
    scratch = [
        pltpu.VMEM((_CH * K,), jnp.int32),
        pltpu.VMEM((_CH * K, D), jnp.float32),
        pltpu.VMEM((_CH * K,), jnp.int32),
        pltpu.VMEM((_CH * K, D), jnp.float32),
        pltpu.VMEM((_CH, D), jnp.float32),
        pltpu.SemaphoreType.DMA,
        pltpu.SemaphoreType.DMA,
        pltpu.SemaphoreType.DMA,
    ]
    if with_p:
        scratch += [
            pltpu.VMEM((_CH,), jnp.int32),
            pltpu.VMEM((_CH, D), jnp.float32),
            pltpu.SemaphoreType.DMA,
        ]

    def k(tabA_ref, tabB_ref, idxA_ref, idxB_ref, *rest):
        if with_p:
            (ptab_ref, pidx_ref, out,
             iA_v, rA_v, iB_v, rB_v, o_v, semA, semB, semO,
             iP_v, rP_v, semP) = rest
        else:
            (out, iA_v, rA_v, iB_v, rB_v, o_v, semA, semB, semO) = rest
        wid = lax.axis_index("s") * _NC + lax.axis_index("c")
        cnt = q + jnp.where(wid < rem, 1, 0)

        def start(ch, tab_ref, idx_ref, idx_v, rows_v, sem):
            pltpu.sync_copy(idx_ref.at[pl.ds(ch * _CH * K, _CH * K)], idx_v)
            pltpu.async_copy(tab_ref.at[idx_v], rows_v, sem)

        def startp(ch):
            pltpu.sync_copy(pidx_ref.at[pl.ds(ch * _CH, _CH)], iP_v)
            pltpu.async_copy(ptab_ref.at[iP_v], rP_v, semP)

        @pl.when(cnt > 0)
        def _():
            start(wid, tabA_ref, idxA_ref, iA_v, rA_v, semA)
            start(wid, tabB_ref, idxB_ref, iB_v, rB_v, semB)
            if with_p:
                startp(wid)

        def reduce_list(rows_v, first):
            for r in range(_CH):
                for c in range(8):
                    sl = pl.ds(c * 16, 16)
                    acc = _tree_sum([rows_v[r * K + kk, sl]
                                     for kk in range(K)])
                    if first:
                        o_v[r, sl] = acc
                    else:
                        o_v[r, sl] = o_v[r, sl] + acc

        @pl.loop(0, cnt)
        def _(i):
            ch = wid + i * _NW
            nxt = ch + _NW

            pltpu.make_async_copy(tabA_ref.at[iA_v], rA_v, semA).wait()

            @pl.when(i > 0)
            def _():
                pltpu.make_async_copy(
                    o_v, out.at[pl.ds((ch - _NW) * _CH, _CH)], semO).wait()

            reduce_list(rA_v, True)

            @pl.when(i + 1 < cnt)
            def _():
                start(nxt, tabA_ref, idxA_ref, iA_v, rA_v, semA)

            pltpu.make_async_copy(tabB_ref.at[iB_v], rB_v, semB).wait()
            reduce_list(rB_v, False)

            @pl.when(i + 1 < cnt)
            def _():
                start(nxt, tabB_ref, idxB_ref, iB_v, rB_v, semB)

            if with_p:
                pltpu.make_async_copy(ptab_ref.at[iP_v], rP_v, semP).wait()
                for r in range(_CH):
                    for c in range(8):
                        sl = pl.ds(c * 16, 16)
                        o_v[r, sl] = o_v[r, sl] + rP_v[r, sl]

                @pl.when(i + 1 < cnt)
                def _():
                    startp(nxt)

            pltpu.async_copy(o_v, out.at[pl.ds(ch * _CH, _CH)], semO)

        @pl.when(cnt > 0)
        def _():
            last = wid + (cnt - 1) * _NW
            pltpu.make_async_copy(
                o_v, out.at[pl.ds(last * _CH, _CH)], semO).wait()

    built = pl.kernel(k,
                      out_type=jax.ShapeDtypeStruct((rows, D), jnp.float32),
                      mesh=_mesh(), scratch_types=scratch)
    if with_p:
        return built(tabA, tabB, idxA, idxB, ptable, pidx)
    return built(tabA, tabB, idxA, idxB)


@jax.jit
def _sc_scatter_add(vals, dst):
    CH = 16
    nch = E // CH
    q, rem = divmod(nch, _NW)
    RB = 16
    nrch = N // RB
    rq, rrem = divmod(nrch, _NS)

    @functools.partial(
        pl.kernel,
        out_type=jax.ShapeDtypeStruct((_NC, N, D), jnp.float32),
        mesh=_mesh(),
        scratch_types=[
            pltpu.VMEM((CH,), jnp.int32),
            pltpu.VMEM((CH, D), jnp.float32),
            pltpu.VMEM((CH,), jnp.int32),
            pltpu.VMEM((CH, D), jnp.float32),
            pltpu.VMEM((RB, D), jnp.float32),
            pltpu.VMEM((RB, D), jnp.float32),
            pltpu.VMEM_SHARED((N, D), jnp.float32),
            pltpu.SemaphoreType.DMA,
            pltpu.SemaphoreType.DMA,
            pltpu.SemaphoreType.DMA,
            pltpu.SemaphoreType.DMA,
            pltpu.SemaphoreType.DMA,
            pltpu.SemaphoreType.DMA,
        ],
    )
    def k(vals_ref, dst_ref, out_ref, idx0_v, rows0_v, idx1_v, rows1_v,
          zbuf, obuf, acc, semI0, semR0, semS0, semI1, semR1, semS1):
        cid = lax.axis_index("c")
        sid = lax.axis_index("s")
        wid = sid * _NC + cid
        rcnt = rq + jnp.where(sid < rrem, 1, 0)

        for r in range(RB):
            for c in range(D // 16):
                zbuf[r, pl.ds(c * 16, 16)] = jnp.zeros((16,), jnp.float32)

        @pl.loop(0, rcnt)
        def _(j):
            pltpu.sync_copy(zbuf, acc.at[pl.ds((sid + j * _NS) * RB, RB)])

        plsc.subcore_barrier()

        cnt = q + jnp.where(wid < rem, 1, 0)
        cnt0 = (cnt + 1) // 2
        cnt1 = cnt // 2
        streams = ((cnt0, 0, idx0_v, rows0_v, semI0, semR0, semS0),
                   (cnt1, 1, idx1_v, rows1_v, semI1, semR1, semS1))

        def load(s, idx_v, rows_v, semI, semR, j):
            ch = wid + (2 * j + s) * _NW
            pltpu.async_copy(dst_ref.at[pl.ds(ch * CH, CH)], idx_v, semI)
            pltpu.async_copy(vals_ref.at[pl.ds(ch * CH, CH)], rows_v, semR)

        for cs, s, idx_v, rows_v, semI, semR, semS in streams:
            @pl.when(cs > 0)
            def _():
                load(s, idx_v, rows_v, semI, semR, 0)

        @pl.loop(0, cnt0)
        def _(j):
            for cs, s, idx_v, rows_v, semI, semR, semS in streams:
                @pl.when(j < cs)
                def _():
                    ch = wid + (2 * j + s) * _NW
                    pltpu.make_async_copy(
                        dst_ref.at[pl.ds(ch * CH, CH)], idx_v, semI).wait()
                    pltpu.make_async_copy(
                        vals_ref.at[pl.ds(ch * CH, CH)], rows_v, semR).wait()
                    pltpu.async_copy(rows_v, acc.at[idx_v], semS, add=True)

            for cs, s, idx_v, rows_v, semI, semR, semS in streams:
                @pl.when(j < cs)
                def _():
                    pltpu.make_async_copy(rows_v, acc.at[idx_v], semS).wait()

                    @pl.when(j + 1 < cs)
                    def _():
                        load(s, idx_v, rows_v, semI, semR, j + 1)

        plsc.subcore_barrier()

        @pl.loop(0, rcnt)
        def _(j):
            off = (sid + j * _NS) * RB
            pltpu.sync_copy(acc.at[pl.ds(off, RB)], obuf)
            pltpu.sync_copy(obuf, out_ref.at[cid, pl.ds(off, RB)])

    return k(vals, dst)


def _dot(a, b):
    return jnp.dot(a, b, preferred_element_type=jnp.float32)


def _tc_pre_y(y, deg, wgy, wgd, wty, w0, w1, bias, *, blk):
    rows = y.shape[0]

    def body(y_ref, d_ref, wgy_r, wgd_r, wty_r, w0_r, w1_r, b_ref,
             ay_ref, wy_ref, z0_ref, z1_ref):
        yb = y_ref[...]
        ay_ref[...] = (_dot(yb, wgy_r[...]) + _dot(yb * d_ref[...], wgd_r[...])
                       + b_ref[...])
        wy_ref[...] = _dot(yb, wty_r[...])
        z0_ref[...] = _dot(yb, w0_r[...])
        z1_ref[...] = _dot(yb, w1_r[...])

    row = pl.BlockSpec((blk, D), lambda i: (i, 0))
    full = pl.BlockSpec((D, D), lambda i: (0, 0))
    vec = pl.BlockSpec((1, D), lambda i: (0, 0))
    return pl.pallas_call(
        body,
        grid=(rows // blk,),
        in_specs=[row, pl.BlockSpec((blk, 1), lambda i: (i, 0)),
                  full, full, full, full, full, vec],
        out_specs=[row, row, row, row],
        out_shape=[jax.ShapeDtypeStruct((rows, D), jnp.float32),
                   jax.ShapeDtypeStruct((rows, D), jnp.float32),
                   jax.ShapeDtypeStruct((rows, D), jnp.float32),
                   jax.ShapeDtypeStruct((rows, D), jnp.float32)],
    )(y, deg, wgy, wgd, wty, w0, w1, bias)


def _tc_pre_x(x, deg, wtx, wtd, w0, w1, wx, bias, *, blk):
    rows = x.shape[0]

    def body(x_ref, d_ref, wtx_r, wtd_r, w0_r, w1_r, wx_r, b_ref,
             ax_ref, u0_ref, u1_ref, zx_ref):
        xb = x_ref[...]
        ax_ref[...] = (_dot(xb, wtx_r[...]) + _dot(xb * d_ref[...], wtd_r[...])
                       + b_ref[...])
        u0_ref[...] = _dot(xb, w0_r[...])
        u1_ref[...] = _dot(xb, w1_r[...])
        zx_ref[...] = _dot(xb, wx_r[...])

    row = pl.BlockSpec((blk, D), lambda i: (i, 0))
    full = pl.BlockSpec((D, D), lambda i: (0, 0))
    vec = pl.BlockSpec((1, D), lambda i: (0, 0))
    return pl.pallas_call(
        body,
        grid=(rows // blk,),
        in_specs=[row, pl.BlockSpec((blk, 1), lambda i: (i, 0)),
                  full, full, full, full, full, vec],
        out_specs=[row, row, row, row],
        out_shape=[jax.ShapeDtypeStruct((rows, D), jnp.float32),
                   jax.ShapeDtypeStruct((rows, D), jnp.float32),
                   jax.ShapeDtypeStruct((rows, D), jnp.float32),
                   jax.ShapeDtypeStruct((rows, D), jnp.float32)],
    )(x, deg, wtx, wtd, w0, w1, wx, bias)


def _relu_half(h):
    col = lax.broadcasted_iota(jnp.int32, h.shape, 1)
    return jnp.where(col >= D // 2, jnp.maximum(h, 0.0), h)


def _tc_stats(parts, *, blk):
    rows = parts[0].shape[0]
    n = len(parts)

    def body(*refs):
        in_refs, h_ref, stats_ref = refs[:n], refs[n], refs[n + 1]
        h = in_refs[0][...].astype(jnp.float32)
        for r in in_refs[1:]:
            h = h + r[...].astype(jnp.float32)
        h = _relu_half(h)
        h_ref[...] = h

        @pl.when(pl.program_id(0) == 0)
        def _():
            stats_ref[...] = jnp.zeros_like(stats_ref)

        stats_ref[0:1, :] = stats_ref[0:1, :] + jnp.sum(h, 0, keepdims=True)
        stats_ref[1:2, :] = stats_ref[1:2, :] + jnp.sum(h * h, 0,
                                                        keepdims=True)

    row = pl.BlockSpec((blk, D), lambda i: (i, 0))
    return pl.pallas_call(
        body,
        grid=(rows // blk,),
        in_specs=[row] * n,
        out_specs=[row, pl.BlockSpec((8, D), lambda i: (0, 0))],
        out_shape=[jax.ShapeDtypeStruct((rows, D), jnp.float32),
                   jax.ShapeDtypeStruct((8, D), jnp.float32)],
    )(*parts)


def _tc_bn(h, stats, s, b, *, blk):
    rows = h.shape[0]
    inv_n = 1.0 / rows

    def body(h_ref, stats_ref, s_ref, b_ref, o_ref):
        m = stats_ref[0:1, :] * inv_n
        v = stats_ref[1:2, :] * inv_n - m * m
        scale = lax.rsqrt(v + 1e-5) * s_ref[...]
        o_ref[...] = (h_ref[...] - m) * scale + b_ref[...]

    row = pl.BlockSpec((blk, D), lambda i: (i, 0))
    vec = pl.BlockSpec((1, D), lambda i: (0, 0))
    return pl.pallas_call(
        body,
        grid=(rows // blk,),
        in_specs=[row, pl.BlockSpec((8, D), lambda i: (0, 0)), vec, vec],
        out_specs=row,
        out_shape=jax.ShapeDtypeStruct((rows, D), jnp.float32),
    )(h, stats, s.reshape(1, D), b.reshape(1, D))


def kernel(x, y, deg_g, deg_lg, pm_pd, g_t, g_tt, lg_t, lg_tt, edge_dst,
           W_tx, b_tx, W_td, b_td, W_ty, b_ty, W_t0, b_t0, W_t1, b_t1,
           W_gy, b_gy, W_gd, b_gd, W_gx, b_gx, W_g0, b_g0, W_g1, b_g1,
           bnx_s, bnx_b, bny_s, bny_b):
    bias_x = (b_tx + b_td + b_t0 + b_t1 + b_ty).reshape(1, D)
    bias_y = (b_gy + b_gd + b_g0 + b_g1 + b_gx).reshape(1, D)

    ay, wy, z0p, z1p = _tc_pre_y(y, deg_lg, W_gy.T, W_gd.T, W_ty.T,
                                 W_g0.T, W_g1.T, bias_y, blk=2000)
    ax, u0p, u1p, zxp = _tc_pre_x(x, deg_g, W_tx.T, W_td.T,
                                  W_t0.T, W_t1.T, W_gx.T, bias_x, blk=2000)

    sx = _sc_gather_fused(u0p, u1p, g_t.reshape(-1), g_tt.reshape(-1),
                          zxp, pm_pd, rows=N, with_p=False)
    sy = _sc_gather_fused(z0p, z1p, lg_t.reshape(-1), lg_tt.reshape(-1),
                          zxp, pm_pd, rows=E, with_p=True)
    py = _sc_scatter_add(wy, edge_dst)

    hx, stx = _tc_stats([ax, sx, py[0], py[1]], blk=2000)
    hy, sty = _tc_stats([ay, sy], blk=2000)
    xn = _tc_bn(hx, stx, bnx_s, bnx_b, blk=2000)
    yn = _tc_bn(hy, sty, bny_s, bny_b, blk=2000)
    return (xn, yn)


# speedup vs baseline: 2.0161x; 2.0161x over previous
import functools

import jax
import jax.numpy as jnp
from jax import lax
from jax.experimental import pallas as pl
from jax.experimental.pallas import tpu as pltpu
from jax.experimental.pallas import tpu_sc as plsc

N = 10000
E = 160000
D = 128
K = 16
DW = D // 2

_info = plsc.get_sparse_core_info()
_NC = _info.num_cores
_NS = _info.num_subcores
_NW = _NC * _NS

_CH = 8


def _mesh():
    return plsc.VectorSubcoreMesh(core_axis_name="c", subcore_axis_name="s")


def _tree_sum(vals):
    vals = list(vals)
    while len(vals) > 1:
        nxt = [vals[i] + vals[i + 1] for i in range(0, len(vals) - 1, 2)]
        if len(vals) % 2:
            nxt.append(vals[-1])
        vals = nxt
    return vals[0]


@functools.partial(jax.jit, static_argnames=("rows", "with_p"))
def _sc_gather_fused(tabA, tabB, idxA, idxB, ptable, pidx, *, rows, with_p):
    nch = rows // _CH
    q, rem = divmod(nch, _NW)
    scratch = [
        pltpu.VMEM((_CH * K,), jnp.int32),
        pltpu.VMEM((_CH * K, D), jnp.float32),
        pltpu.VMEM((_CH * K,), jnp.int32),
        pltpu.VMEM((_CH * K, D), jnp.float32),
        pltpu.VMEM((_CH, D), jnp.float32),
        pltpu.SemaphoreType.DMA,
        pltpu.SemaphoreType.DMA,
        pltpu.SemaphoreType.DMA,
    ]
    if with_p:
        scratch += [
            pltpu.VMEM((_CH,), jnp.int32),
            pltpu.VMEM((_CH, D), jnp.float32),
            pltpu.SemaphoreType.DMA,
        ]

    def k(tabA_ref, tabB_ref, idxA_ref, idxB_ref, *rest):
        if with_p:
            (ptab_ref, pidx_ref, out,
             iA_v, rA_v, iB_v, rB_v, o_v, semA, semB, semO,
             iP_v, rP_v, semP) = rest
        else:
            (out, iA_v, rA_v, iB_v, rB_v, o_v, semA, semB, semO) = rest
        wid = lax.axis_index("s") * _NC + lax.axis_index("c")
        cnt = q + jnp.where(wid < rem, 1, 0)

        def start(ch, tab_ref, idx_ref, idx_v, rows_v, sem):
            pltpu.sync_copy(idx_ref.at[pl.ds(ch * _CH * K, _CH * K)], idx_v)
            pltpu.async_copy(tab_ref.at[idx_v], rows_v, sem)

        def startp(ch):
            pltpu.sync_copy(pidx_ref.at[pl.ds(ch * _CH, _CH)], iP_v)
            pltpu.async_copy(ptab_ref.at[iP_v], rP_v, semP)

        @pl.when(cnt > 0)
        def _():
            start(wid, tabA_ref, idxA_ref, iA_v, rA_v, semA)
            start(wid, tabB_ref, idxB_ref, iB_v, rB_v, semB)
            if with_p:
                startp(wid)

        def reduce_list(rows_v, first):
            @pl.loop(0, _CH)
            def _(r):
                for c in range(8):
                    sl = pl.ds(c * 16, 16)
                    acc = rows_v[r * K, sl]
                    for kk in range(1, K):
                        acc = acc + rows_v[r * K + kk, sl]
                    if first:
                        o_v[r, sl] = acc
                    else:
                        o_v[r, sl] = o_v[r, sl] + acc

        @pl.loop(0, cnt)
        def _(i):
            ch = wid + i * _NW
            nxt = ch + _NW

            pltpu.make_async_copy(tabA_ref.at[iA_v], rA_v, semA).wait()

            @pl.when(i > 0)
            def _():
                pltpu.make_async_copy(
                    o_v, out.at[pl.ds((ch - _NW) * _CH, _CH)], semO).wait()

            reduce_list(rA_v, True)

            @pl.when(i + 1 < cnt)
            def _():
                start(nxt, tabA_ref, idxA_ref, iA_v, rA_v, semA)

            pltpu.make_async_copy(tabB_ref.at[iB_v], rB_v, semB).wait()
            reduce_list(rB_v, False)

            @pl.when(i + 1 < cnt)
            def _():
                start(nxt, tabB_ref, idxB_ref, iB_v, rB_v, semB)

            if with_p:
                pltpu.make_async_copy(ptab_ref.at[iP_v], rP_v, semP).wait()

                @pl.loop(0, _CH)
                def _(r):
                    for c in range(8):
                        sl = pl.ds(c * 16, 16)
                        o_v[r, sl] = o_v[r, sl] + rP_v[r, sl]

                @pl.when(i + 1 < cnt)
                def _():
                    startp(nxt)

            pltpu.async_copy(o_v, out.at[pl.ds(ch * _CH, _CH)], semO)

        @pl.when(cnt > 0)
        def _():
            last = wid + (cnt - 1) * _NW
            pltpu.make_async_copy(
                o_v, out.at[pl.ds(last * _CH, _CH)], semO).wait()

    built = pl.kernel(k,
                      out_type=jax.ShapeDtypeStruct((rows, D), jnp.float32),
                      mesh=_mesh(), scratch_types=scratch)
    if with_p:
        return built(tabA, tabB, idxA, idxB, ptable, pidx)
    return built(tabA, tabB, idxA, idxB)


_IB = 25


@jax.jit
def _sc_gather_e(tabA, tabB, idxA, idxB, ptable, pidx):
    cnt = E // _CH // _NW
    nb = cnt // _IB
    BR = _IB * _CH

    @functools.partial(
        pl.kernel,
        out_type=jax.ShapeDtypeStruct((E, D), jnp.float32),
        mesh=_mesh(),
        scratch_types=[
            pltpu.VMEM((_IB * _CH * K,), jnp.int32),
            pltpu.VMEM((_IB * _CH * K,), jnp.int32),
            pltpu.VMEM((BR,), jnp.int32),
            pltpu.VMEM((_CH * K, D), jnp.float32),
            pltpu.VMEM((_CH * K, D), jnp.float32),
            pltpu.VMEM((BR, D), jnp.float32),
            pltpu.VMEM((BR, D), jnp.float32),
            pltpu.SemaphoreType.DMA,
            pltpu.SemaphoreType.DMA,
            pltpu.SemaphoreType.DMA,
            pltpu.SemaphoreType.DMA,
        ],
    )
    def k(tabA_ref, tabB_ref, idxA_ref, idxB_ref, ptab_ref, pidx_ref, out,
          iA_v, iB_v, iP_v, rA_v, rB_v, rP_v, o_v, semA, semB, semP, semO):
        wid = lax.axis_index("s") * _NC + lax.axis_index("c")
        s0 = wid * cnt

        def startA(t):
            pltpu.async_copy(tabA_ref.at[iA_v.at[pl.ds(t * _CH * K,
                                                       _CH * K)]],
                             rA_v, semA)

        def startB(t):
            pltpu.async_copy(tabB_ref.at[iB_v.at[pl.ds(t * _CH * K,
                                                       _CH * K)]],
                             rB_v, semB)

        @pl.loop(0, nb)
        def _(b):
            bc = s0 + b * _IB

            @pl.when(b > 0)
            def _():
                pltpu.make_async_copy(
                    o_v, out.at[pl.ds((bc - _IB) * _CH, BR)], semO).wait()

            pltpu.sync_copy(idxA_ref.at[pl.ds(bc * _CH * K, _IB * _CH * K)],
                            iA_v)
            pltpu.sync_copy(idxB_ref.at[pl.ds(bc * _CH * K, _IB * _CH * K)],
                            iB_v)
            pltpu.sync_copy(pidx_ref.at[pl.ds(bc * _CH, BR)], iP_v)
            startA(0)
            startB(0)
            pltpu.async_copy(ptab_ref.at[iP_v.at[pl.ds(0, 128)]],
                             rP_v.at[pl.ds(0, 128)], semP)
            pltpu.async_copy(ptab_ref.at[iP_v.at[pl.ds(128, BR - 128)]],
                             rP_v.at[pl.ds(128, BR - 128)], semP)

            @pl.loop(0, _IB)
            def _(t):
                pltpu.make_async_copy(
                    tabA_ref.at[iA_v.at[pl.ds(t * _CH * K, _CH * K)]],
                    rA_v, semA).wait()

                @pl.loop(0, _CH)
                def _(r):
                    for c in range(8):
                        sl = pl.ds(c * 16, 16)
                        acc = rA_v[r * K, sl]
                        for kk in range(1, K):
                            acc = acc + rA_v[r * K + kk, sl]
                        o_v[t * _CH + r, sl] = acc

                @pl.when(t + 1 < _IB)
                def _():
                    startA(t + 1)

                pltpu.make_async_copy(
                    tabB_ref.at[iB_v.at[pl.ds(t * _CH * K, _CH * K)]],
                    rB_v, semB).wait()

                @pl.loop(0, _CH)
                def _(r):
                    for c in range(8):
                        sl = pl.ds(c * 16, 16)
                        acc = rB_v[r * K, sl]
                        for kk in range(1, K):
                            acc = acc + rB_v[r * K + kk, sl]
                        o_v[t * _CH + r, sl] = o_v[t * _CH + r, sl] + acc

                @pl.when(t + 1 < _IB)
                def _():
                    startB(t + 1)

                @pl.when(t == 0)
                def _():
                    pltpu.make_async_copy(
                        ptab_ref.at[iP_v.at[pl.ds(128, BR - 128)]],
                        rP_v.at[pl.ds(128, BR - 128)], semP).wait()
                    pltpu.make_async_copy(
                        ptab_ref.at[iP_v.at[pl.ds(0, 128)]],
                        rP_v.at[pl.ds(0, 128)], semP).wait()

                @pl.loop(0, _CH)
                def _(r):
                    for c in range(8):
                        sl = pl.ds(c * 16, 16)
                        o_v[t * _CH + r, sl] = (o_v[t * _CH + r, sl]
                                                + rP_v[t * _CH + r, sl])

            pltpu.async_copy(o_v, out.at[pl.ds(bc * _CH, BR)], semO)

        pltpu.make_async_copy(
            o_v, out.at[pl.ds((s0 + (nb - 1) * _IB) * _CH, BR)], semO).wait()

    return k(tabA, tabB, idxA, idxB, ptable, pidx)


@jax.jit
def _sc_scatter_add(vals, dst):
    CH = 16
    nch = E // CH
    q, rem = divmod(nch, _NW)
    RB = 16
    nrch = N // RB
    rq, rrem = divmod(nrch, _NS)

    @functools.partial(
        pl.kernel,
        out_type=jax.ShapeDtypeStruct((_NC, N, D), jnp.float32),
        mesh=_mesh(),
        scratch_types=[
            pltpu.VMEM((CH,), jnp.int32),
            pltpu.VMEM((CH, D), jnp.float32),
            pltpu.VMEM((CH,), jnp.int32),
            pltpu.VMEM((CH, D), jnp.float32),
            pltpu.VMEM((RB, D), jnp.float32),
            pltpu.VMEM((RB, D), jnp.float32),
            pltpu.VMEM_SHARED((N, D), jnp.float32),
            pltpu.SemaphoreType.DMA,
            pltpu.SemaphoreType.DMA,
            pltpu.SemaphoreType.DMA,
            pltpu.SemaphoreType.DMA,
            pltpu.SemaphoreType.DMA,
            pltpu.SemaphoreType.DMA,
        ],
    )
    def k(vals_ref, dst_ref, out_ref, idx0_v, rows0_v, idx1_v, rows1_v,
          zbuf, obuf, acc, semI0, semR0, semS0, semI1, semR1, semS1):
        cid = lax.axis_index("c")
        sid = lax.axis_index("s")
        wid = sid * _NC + cid
        rcnt = rq + jnp.where(sid < rrem, 1, 0)

        for r in range(RB):
            for c in range(D // 16):
                zbuf[r, pl.ds(c * 16, 16)] = jnp.zeros((16,), jnp.float32)

        @pl.loop(0, rcnt)
        def _(j):
            pltpu.sync_copy(zbuf, acc.at[pl.ds((sid + j * _NS) * RB, RB)])

        plsc.subcore_barrier()

        cnt = q + jnp.where(wid < rem, 1, 0)
        cnt0 = (cnt + 1) // 2
        cnt1 = cnt // 2
        streams = ((cnt0, 0, idx0_v, rows0_v, semI0, semR0, semS0),
                   (cnt1, 1, idx1_v, rows1_v, semI1, semR1, semS1))

        def load(s, idx_v, rows_v, semI, semR, j):
            ch = wid + (2 * j + s) * _NW
            pltpu.async_copy(dst_ref.at[pl.ds(ch * CH, CH)], idx_v, semI)
            pltpu.async_copy(vals_ref.at[pl.ds(ch * CH, CH)], rows_v, semR)

        for cs, s, idx_v, rows_v, semI, semR, semS in streams:
            @pl.when(cs > 0)
            def _():
                load(s, idx_v, rows_v, semI, semR, 0)

        @pl.loop(0, cnt0)
        def _(j):
            for cs, s, idx_v, rows_v, semI, semR, semS in streams:
                @pl.when(j < cs)
                def _():
                    ch = wid + (2 * j + s) * _NW
                    pltpu.make_async_copy(
                        dst_ref.at[pl.ds(ch * CH, CH)], idx_v, semI).wait()
                    pltpu.make_async_copy(
                        vals_ref.at[pl.ds(ch * CH, CH)], rows_v, semR).wait()
                    pltpu.async_copy(rows_v, acc.at[idx_v], semS, add=True)

            for cs, s, idx_v, rows_v, semI, semR, semS in streams:
                @pl.when(j < cs)
                def _():
                    pltpu.make_async_copy(rows_v, acc.at[idx_v], semS).wait()

                    @pl.when(j + 1 < cs)
                    def _():
                        load(s, idx_v, rows_v, semI, semR, j + 1)

        plsc.subcore_barrier()

        @pl.loop(0, rcnt)
        def _(j):
            off = (sid + j * _NS) * RB
            pltpu.sync_copy(acc.at[pl.ds(off, RB)], obuf)
            pltpu.sync_copy(obuf, out_ref.at[cid, pl.ds(off, RB)])

    return k(vals, dst)


def _dot(a, b):
    return jnp.dot(a, b, preferred_element_type=jnp.float32)


def _tc_pre_y(y, deg, wgy, wgd, wty, w0, w1, bias, *, blk):
    rows = y.shape[0]

    def body(y_ref, d_ref, wgy_r, wgd_r, wty_r, w0_r, w1_r, b_ref,
             ay_ref, wy_ref, z0_ref, z1_ref):
        yb = y_ref[...]
        ay_ref[...] = (_dot(yb, wgy_r[...]) + _dot(yb * d_ref[...], wgd_r[...])
                       + b_ref[...])
        wy_ref[...] = _dot(yb, wty_r[...])
        z0_ref[...] = _dot(yb, w0_r[...])
        z1_ref[...] = _dot(yb, w1_r[...])

    row = pl.BlockSpec((blk, D), lambda i: (i, 0))
    full = pl.BlockSpec((D, D), lambda i: (0, 0))
    vec = pl.BlockSpec((1, D), lambda i: (0, 0))
    return pl.pallas_call(
        body,
        grid=(rows // blk,),
        in_specs=[row, pl.BlockSpec((blk, 1), lambda i: (i, 0)),
                  full, full, full, full, full, vec],
        out_specs=[row, row, row, row],
        out_shape=[jax.ShapeDtypeStruct((rows, D), jnp.float32),
                   jax.ShapeDtypeStruct((rows, D), jnp.float32),
                   jax.ShapeDtypeStruct((rows, D), jnp.float32),
                   jax.ShapeDtypeStruct((rows, D), jnp.float32)],
    )(y, deg, wgy, wgd, wty, w0, w1, bias)


def _tc_pre_x(x, deg, wtx, wtd, w0, w1, wx, bias, *, blk):
    rows = x.shape[0]

    def body(x_ref, d_ref, wtx_r, wtd_r, w0_r, w1_r, wx_r, b_ref,
             ax_ref, u0_ref, u1_ref, zx_ref):
        xb = x_ref[...]
        ax_ref[...] = (_dot(xb, wtx_r[...]) + _dot(xb * d_ref[...], wtd_r[...])
                       + b_ref[...])
        u0_ref[...] = _dot(xb, w0_r[...])
        u1_ref[...] = _dot(xb, w1_r[...])
        zx_ref[...] = _dot(xb, wx_r[...])

    row = pl.BlockSpec((blk, D), lambda i: (i, 0))
    full = pl.BlockSpec((D, D), lambda i: (0, 0))
    vec = pl.BlockSpec((1, D), lambda i: (0, 0))
    return pl.pallas_call(
        body,
        grid=(rows // blk,),
        in_specs=[row, pl.BlockSpec((blk, 1), lambda i: (i, 0)),
                  full, full, full, full, full, vec],
        out_specs=[row, row, row, row],
        out_shape=[jax.ShapeDtypeStruct((rows, D), jnp.float32),
                   jax.ShapeDtypeStruct((rows, D), jnp.float32),
                   jax.ShapeDtypeStruct((rows, D), jnp.float32),
                   jax.ShapeDtypeStruct((rows, D), jnp.float32)],
    )(x, deg, wtx, wtd, w0, w1, wx, bias)


def _relu_half(h):
    col = lax.broadcasted_iota(jnp.int32, h.shape, 1)
    return jnp.where(col >= D // 2, jnp.maximum(h, 0.0), h)


def _tc_stats(parts, *, blk):
    rows = parts[0].shape[0]
    n = len(parts)

    def body(*refs):
        in_refs, h_ref, stats_ref = refs[:n], refs[n], refs[n + 1]
        h = in_refs[0][...].astype(jnp.float32)
        for r in in_refs[1:]:
            h = h + r[...].astype(jnp.float32)
        h = _relu_half(h)
        h_ref[...] = h

        @pl.when(pl.program_id(0) == 0)
        def _():
            stats_ref[...] = jnp.zeros_like(stats_ref)

        stats_ref[0:1, :] = stats_ref[0:1, :] + jnp.sum(h, 0, keepdims=True)
        stats_ref[1:2, :] = stats_ref[1:2, :] + jnp.sum(h * h, 0,
                                                        keepdims=True)

    row = pl.BlockSpec((blk, D), lambda i: (i, 0))
    return pl.pallas_call(
        body,
        grid=(rows // blk,),
        in_specs=[row] * n,
        out_specs=[row, pl.BlockSpec((8, D), lambda i: (0, 0))],
        out_shape=[jax.ShapeDtypeStruct((rows, D), jnp.float32),
                   jax.ShapeDtypeStruct((8, D), jnp.float32)],
    )(*parts)


def _tc_bn(h, stats, s, b, *, blk):
    rows = h.shape[0]
    inv_n = 1.0 / rows

    def body(h_ref, stats_ref, s_ref, b_ref, o_ref):
        m = stats_ref[0:1, :] * inv_n
        v = stats_ref[1:2, :] * inv_n - m * m
        scale = lax.rsqrt(v + 1e-5) * s_ref[...]
        o_ref[...] = (h_ref[...] - m) * scale + b_ref[...]

    row = pl.BlockSpec((blk, D), lambda i: (i, 0))
    vec = pl.BlockSpec((1, D), lambda i: (0, 0))
    return pl.pallas_call(
        body,
        grid=(rows // blk,),
        in_specs=[row, pl.BlockSpec((8, D), lambda i: (0, 0)), vec, vec],
        out_specs=row,
        out_shape=jax.ShapeDtypeStruct((rows, D), jnp.float32),
    )(h, stats, s.reshape(1, D), b.reshape(1, D))


def kernel(x, y, deg_g, deg_lg, pm_pd, g_t, g_tt, lg_t, lg_tt, edge_dst,
           W_tx, b_tx, W_td, b_td, W_ty, b_ty, W_t0, b_t0, W_t1, b_t1,
           W_gy, b_gy, W_gd, b_gd, W_gx, b_gx, W_g0, b_g0, W_g1, b_g1,
           bnx_s, bnx_b, bny_s, bny_b):
    bias_x = (b_tx + b_td + b_t0 + b_t1 + b_ty).reshape(1, D)
    bias_y = (b_gy + b_gd + b_g0 + b_g1 + b_gx).reshape(1, D)

    ay, wy, z0p, z1p = _tc_pre_y(y, deg_lg, W_gy.T, W_gd.T, W_ty.T,
                                 W_g0.T, W_g1.T, bias_y, blk=2000)
    ax, u0p, u1p, zxp = _tc_pre_x(x, deg_g, W_tx.T, W_td.T,
                                  W_t0.T, W_t1.T, W_gx.T, bias_x, blk=2000)

    sx = _sc_gather_fused(u0p, u1p, g_t.reshape(-1), g_tt.reshape(-1),
                          zxp, pm_pd, rows=N, with_p=False)
    sy = _sc_gather_e(z0p, z1p, lg_t.reshape(-1), lg_tt.reshape(-1),
                      zxp, pm_pd)
    py = _sc_scatter_add(wy, edge_dst)

    hx, stx = _tc_stats([ax, sx, py[0], py[1]], blk=2000)
    hy, sty = _tc_stats([ay, sy], blk=2000)
    xn = _tc_bn(hx, stx, bnx_s, bnx_b, blk=2000)
    yn = _tc_bn(hy, sty, bny_s, bny_b, blk=2000)
    return (xn, yn)


# speedup vs baseline: 2.3752x; 1.1781x over previous
import functools

import jax
import jax.numpy as jnp
from jax import lax
from jax.experimental import pallas as pl
from jax.experimental.pallas import tpu as pltpu
from jax.experimental.pallas import tpu_sc as plsc

N = 10000
E = 160000
D = 128
K = 16
DW = D // 2

_info = plsc.get_sparse_core_info()
_NC = _info.num_cores
_NS = _info.num_subcores
_NW = _NC * _NS

_CH = 8


def _mesh():
    return plsc.VectorSubcoreMesh(core_axis_name="c", subcore_axis_name="s")


def _acc4(load, n):
    a = [load(0), load(1), load(2), load(3)]
    for kk in range(4, n):
        a[kk % 4] = a[kk % 4] + load(kk)
    return (a[0] + a[1]) + (a[2] + a[3])


def _tree_sum(vals):
    vals = list(vals)
    while len(vals) > 1:
        nxt = [vals[i] + vals[i + 1] for i in range(0, len(vals) - 1, 2)]
        if len(vals) % 2:
            nxt.append(vals[-1])
        vals = nxt
    return vals[0]


@functools.partial(jax.jit, static_argnames=("rows", "with_p"))
def _sc_gather_fused(tabA, tabB, idxA, idxB, ptable, pidx, *, rows, with_p):
    nch = rows // _CH
    q, rem = divmod(nch, _NW)
    scratch = [
        pltpu.VMEM((_CH * K,), jnp.int32),
        pltpu.VMEM((_CH * K, D), jnp.float32),
        pltpu.VMEM((_CH * K,), jnp.int32),
        pltpu.VMEM((_CH * K, D), jnp.float32),
        pltpu.VMEM((_CH, D), jnp.float32),
        pltpu.SemaphoreType.DMA,
        pltpu.SemaphoreType.DMA,
        pltpu.SemaphoreType.DMA,
    ]
    if with_p:
        scratch += [
            pltpu.VMEM((_CH,), jnp.int32),
            pltpu.VMEM((_CH, D), jnp.float32),
            pltpu.SemaphoreType.DMA,
        ]

    def k(tabA_ref, tabB_ref, idxA_ref, idxB_ref, *rest):
        if with_p:
            (ptab_ref, pidx_ref, out,
             iA_v, rA_v, iB_v, rB_v, o_v, semA, semB, semO,
             iP_v, rP_v, semP) = rest
        else:
            (out, iA_v, rA_v, iB_v, rB_v, o_v, semA, semB, semO) = rest
        wid = lax.axis_index("s") * _NC + lax.axis_index("c")
        cnt = q + jnp.where(wid < rem, 1, 0)

        def start(ch, tab_ref, idx_ref, idx_v, rows_v, sem):
            pltpu.sync_copy(idx_ref.at[pl.ds(ch * _CH * K, _CH * K)], idx_v)
            pltpu.async_copy(tab_ref.at[idx_v], rows_v, sem)

        def startp(ch):
            pltpu.sync_copy(pidx_ref.at[pl.ds(ch * _CH, _CH)], iP_v)
            pltpu.async_copy(ptab_ref.at[iP_v], rP_v, semP)

        @pl.when(cnt > 0)
        def _():
            start(wid, tabA_ref, idxA_ref, iA_v, rA_v, semA)
            start(wid, tabB_ref, idxB_ref, iB_v, rB_v, semB)
            if with_p:
                startp(wid)

        def reduce_list(rows_v, first):
            @pl.loop(0, _CH)
            def _(r):
                for c in range(8):
                    sl = pl.ds(c * 16, 16)
                    acc = _acc4(lambda kk: rows_v[r * K + kk, sl], K)
                    if first:
                        o_v[r, sl] = acc
                    else:
                        o_v[r, sl] = o_v[r, sl] + acc

        @pl.loop(0, cnt)
        def _(i):
            ch = wid + i * _NW
            nxt = ch + _NW

            pltpu.make_async_copy(tabA_ref.at[iA_v], rA_v, semA).wait()

            @pl.when(i > 0)
            def _():
                pltpu.make_async_copy(
                    o_v, out.at[pl.ds((ch - _NW) * _CH, _CH)], semO).wait()

            reduce_list(rA_v, True)

            @pl.when(i + 1 < cnt)
            def _():
                start(nxt, tabA_ref, idxA_ref, iA_v, rA_v, semA)

            pltpu.make_async_copy(tabB_ref.at[iB_v], rB_v, semB).wait()
            reduce_list(rB_v, False)

            @pl.when(i + 1 < cnt)
            def _():
                start(nxt, tabB_ref, idxB_ref, iB_v, rB_v, semB)

            if with_p:
                pltpu.make_async_copy(ptab_ref.at[iP_v], rP_v, semP).wait()

                @pl.loop(0, _CH)
                def _(r):
                    for c in range(8):
                        sl = pl.ds(c * 16, 16)
                        o_v[r, sl] = o_v[r, sl] + rP_v[r, sl]

                @pl.when(i + 1 < cnt)
                def _():
                    startp(nxt)

            pltpu.async_copy(o_v, out.at[pl.ds(ch * _CH, _CH)], semO)

        @pl.when(cnt > 0)
        def _():
            last = wid + (cnt - 1) * _NW
            pltpu.make_async_copy(
                o_v, out.at[pl.ds(last * _CH, _CH)], semO).wait()

    built = pl.kernel(k,
                      out_type=jax.ShapeDtypeStruct((rows, D), jnp.float32),
                      mesh=_mesh(), scratch_types=scratch)
    if with_p:
        return built(tabA, tabB, idxA, idxB, ptable, pidx)
    return built(tabA, tabB, idxA, idxB)


_IB = 25


@jax.jit
def _sc_gather_e(tabA, tabB, idxA, idxB, ptable, pidx):
    cnt = E // _CH // _NW
    nb = cnt // _IB
    BR = _IB * _CH

    @functools.partial(
        pl.kernel,
        out_type=jax.ShapeDtypeStruct((E, D), jnp.float32),
        mesh=_mesh(),
        scratch_types=[
            pltpu.VMEM((_IB * _CH * K,), jnp.int32),
            pltpu.VMEM((_IB * _CH * K,), jnp.int32),
            pltpu.VMEM((BR,), jnp.int32),
            pltpu.VMEM((_CH * K, D), jnp.float32),
            pltpu.VMEM((_CH * K, D), jnp.float32),
            pltpu.VMEM((BR, D), jnp.float32),
            pltpu.VMEM((BR, D), jnp.float32),
            pltpu.SemaphoreType.DMA,
            pltpu.SemaphoreType.DMA,
            pltpu.SemaphoreType.DMA,
            pltpu.SemaphoreType.DMA,
        ],
    )
    def k(tabA_ref, tabB_ref, idxA_ref, idxB_ref, ptab_ref, pidx_ref, out,
          iA_v, iB_v, iP_v, rA_v, rB_v, rP_v, o_v, semA, semB, semP, semO):
        wid = lax.axis_index("s") * _NC + lax.axis_index("c")
        s0 = wid * cnt

        def startA(t):
            pltpu.async_copy(tabA_ref.at[iA_v.at[pl.ds(t * _CH * K,
                                                       _CH * K)]],
                             rA_v, semA)

        def startB(t):
            pltpu.async_copy(tabB_ref.at[iB_v.at[pl.ds(t * _CH * K,
                                                       _CH * K)]],
                             rB_v, semB)

        @pl.loop(0, nb)
        def _(b):
            bc = s0 + b * _IB

            @pl.when(b > 0)
            def _():
                pltpu.make_async_copy(
                    o_v, out.at[pl.ds((bc - _IB) * _CH, BR)], semO).wait()

            pltpu.sync_copy(idxA_ref.at[pl.ds(bc * _CH * K, _IB * _CH * K)],
                            iA_v)
            pltpu.sync_copy(idxB_ref.at[pl.ds(bc * _CH * K, _IB * _CH * K)],
                            iB_v)
            pltpu.sync_copy(pidx_ref.at[pl.ds(bc * _CH, BR)], iP_v)
            startA(0)
            startB(0)
            pltpu.async_copy(ptab_ref.at[iP_v.at[pl.ds(0, 128)]],
                             rP_v.at[pl.ds(0, 128)], semP)
            pltpu.async_copy(ptab_ref.at[iP_v.at[pl.ds(128, BR - 128)]],
                             rP_v.at[pl.ds(128, BR - 128)], semP)

            @pl.loop(0, _IB)
            def _(t):
                pltpu.make_async_copy(
                    tabA_ref.at[iA_v.at[pl.ds(t * _CH * K, _CH * K)]],
                    rA_v, semA).wait()

                @pl.loop(0, _CH)
                def _(r):
                    for c in range(8):
                        sl = pl.ds(c * 16, 16)
                        acc = _acc4(lambda kk: rA_v[r * K + kk, sl], K)
                        o_v[t * _CH + r, sl] = acc

                @pl.when(t + 1 < _IB)
                def _():
                    startA(t + 1)

                pltpu.make_async_copy(
                    tabB_ref.at[iB_v.at[pl.ds(t * _CH * K, _CH * K)]],
                    rB_v, semB).wait()

                @pl.loop(0, _CH)
                def _(r):
                    for c in range(8):
                        sl = pl.ds(c * 16, 16)
                        acc = _acc4(lambda kk: rB_v[r * K + kk, sl], K)
                        o_v[t * _CH + r, sl] = o_v[t * _CH + r, sl] + acc

                @pl.when(t + 1 < _IB)
                def _():
                    startB(t + 1)

                @pl.when(t == 0)
                def _():
                    pltpu.make_async_copy(
                        ptab_ref.at[iP_v.at[pl.ds(128, BR - 128)]],
                        rP_v.at[pl.ds(128, BR - 128)], semP).wait()
                    pltpu.make_async_copy(
                        ptab_ref.at[iP_v.at[pl.ds(0, 128)]],
                        rP_v.at[pl.ds(0, 128)], semP).wait()

                @pl.loop(0, _CH)
                def _(r):
                    for c in range(8):
                        sl = pl.ds(c * 16, 16)
                        o_v[t * _CH + r, sl] = (o_v[t * _CH + r, sl]
                                                + rP_v[t * _CH + r, sl])

            pltpu.async_copy(o_v, out.at[pl.ds(bc * _CH, BR)], semO)

        pltpu.make_async_copy(
            o_v, out.at[pl.ds((s0 + (nb - 1) * _IB) * _CH, BR)], semO).wait()

    return k(tabA, tabB, idxA, idxB, ptable, pidx)


@jax.jit
def _sc_scatter_add(vals, dst):
    CH = 16
    nch = E // CH
    q, rem = divmod(nch, _NW)
    RB = 16
    nrch = N // RB
    rq, rrem = divmod(nrch, _NS)

    @functools.partial(
        pl.kernel,
        out_type=jax.ShapeDtypeStruct((_NC, N, D), jnp.float32),
        mesh=_mesh(),
        scratch_types=[
            pltpu.VMEM((CH,), jnp.int32),
            pltpu.VMEM((CH, D), jnp.float32),
            pltpu.VMEM((CH,), jnp.int32),
            pltpu.VMEM((CH, D), jnp.float32),
            pltpu.VMEM((RB, D), jnp.float32),
            pltpu.VMEM((RB, D), jnp.float32),
            pltpu.VMEM_SHARED((N, D), jnp.float32),
            pltpu.SemaphoreType.DMA,
            pltpu.SemaphoreType.DMA,
            pltpu.SemaphoreType.DMA,
            pltpu.SemaphoreType.DMA,
            pltpu.SemaphoreType.DMA,
            pltpu.SemaphoreType.DMA,
        ],
    )
    def k(vals_ref, dst_ref, out_ref, idx0_v, rows0_v, idx1_v, rows1_v,
          zbuf, obuf, acc, semI0, semR0, semS0, semI1, semR1, semS1):
        cid = lax.axis_index("c")
        sid = lax.axis_index("s")
        wid = sid * _NC + cid
        rcnt = rq + jnp.where(sid < rrem, 1, 0)

        for r in range(RB):
            for c in range(D // 16):
                zbuf[r, pl.ds(c * 16, 16)] = jnp.zeros((16,), jnp.float32)

        @pl.loop(0, rcnt)
        def _(j):
            pltpu.sync_copy(zbuf, acc.at[pl.ds((sid + j * _NS) * RB, RB)])

        plsc.subcore_barrier()

        cnt = q + jnp.where(wid < rem, 1, 0)
        cnt0 = (cnt + 1) // 2
        cnt1 = cnt // 2
        streams = ((cnt0, 0, idx0_v, rows0_v, semI0, semR0, semS0),
                   (cnt1, 1, idx1_v, rows1_v, semI1, semR1, semS1))

        def load(s, idx_v, rows_v, semI, semR, j):
            ch = wid + (2 * j + s) * _NW
            pltpu.async_copy(dst_ref.at[pl.ds(ch * CH, CH)], idx_v, semI)
            pltpu.async_copy(vals_ref.at[pl.ds(ch * CH, CH)], rows_v, semR)

        for cs, s, idx_v, rows_v, semI, semR, semS in streams:
            @pl.when(cs > 0)
            def _():
                load(s, idx_v, rows_v, semI, semR, 0)

        @pl.loop(0, cnt0)
        def _(j):
            for cs, s, idx_v, rows_v, semI, semR, semS in streams:
                @pl.when(j < cs)
                def _():
                    ch = wid + (2 * j + s) * _NW
                    pltpu.make_async_copy(
                        dst_ref.at[pl.ds(ch * CH, CH)], idx_v, semI).wait()
                    pltpu.make_async_copy(
                        vals_ref.at[pl.ds(ch * CH, CH)], rows_v, semR).wait()
                    pltpu.async_copy(rows_v, acc.at[idx_v], semS, add=True)

            for cs, s, idx_v, rows_v, semI, semR, semS in streams:
                @pl.when(j < cs)
                def _():
                    pltpu.make_async_copy(rows_v, acc.at[idx_v], semS).wait()

                    @pl.when(j + 1 < cs)
                    def _():
                        load(s, idx_v, rows_v, semI, semR, j + 1)

        plsc.subcore_barrier()

        @pl.loop(0, rcnt)
        def _(j):
            off = (sid + j * _NS) * RB
            pltpu.sync_copy(acc.at[pl.ds(off, RB)], obuf)
            pltpu.sync_copy(obuf, out_ref.at[cid, pl.ds(off, RB)])

    return k(vals, dst)


def _dot(a, b):
    return jnp.dot(a, b, preferred_element_type=jnp.float32)


def _tc_pre_y(y, deg, wgy, wgd, wty, w0, w1, bias, *, blk):
    rows = y.shape[0]

    def body(y_ref, d_ref, wgy_r, wgd_r, wty_r, w0_r, w1_r, b_ref,
             ay_ref, wy_ref, z0_ref, z1_ref):
        yb = y_ref[...]
        ay_ref[...] = (_dot(yb, wgy_r[...]) + _dot(yb * d_ref[...], wgd_r[...])
                       + b_ref[...])
        wy_ref[...] = _dot(yb, wty_r[...])
        z0_ref[...] = _dot(yb, w0_r[...])
        z1_ref[...] = _dot(yb, w1_r[...])

    row = pl.BlockSpec((blk, D), lambda i: (i, 0))
    full = pl.BlockSpec((D, D), lambda i: (0, 0))
    vec = pl.BlockSpec((1, D), lambda i: (0, 0))
    return pl.pallas_call(
        body,
        grid=(rows // blk,),
        in_specs=[row, pl.BlockSpec((blk, 1), lambda i: (i, 0)),
                  full, full, full, full, full, vec],
        out_specs=[row, row, row, row],
        out_shape=[jax.ShapeDtypeStruct((rows, D), jnp.float32),
                   jax.ShapeDtypeStruct((rows, D), jnp.float32),
                   jax.ShapeDtypeStruct((rows, D), jnp.float32),
                   jax.ShapeDtypeStruct((rows, D), jnp.float32)],
    )(y, deg, wgy, wgd, wty, w0, w1, bias)


def _tc_pre_x(x, deg, wtx, wtd, w0, w1, wx, bias, *, blk):
    rows = x.shape[0]

    def body(x_ref, d_ref, wtx_r, wtd_r, w0_r, w1_r, wx_r, b_ref,
             ax_ref, u0_ref, u1_ref, zx_ref):
        xb = x_ref[...]
        ax_ref[...] = (_dot(xb, wtx_r[...]) + _dot(xb * d_ref[...], wtd_r[...])
                       + b_ref[...])
        u0_ref[...] = _dot(xb, w0_r[...])
        u1_ref[...] = _dot(xb, w1_r[...])
        zx_ref[...] = _dot(xb, wx_r[...])

    row = pl.BlockSpec((blk, D), lambda i: (i, 0))
    full = pl.BlockSpec((D, D), lambda i: (0, 0))
    vec = pl.BlockSpec((1, D), lambda i: (0, 0))
    return pl.pallas_call(
        body,
        grid=(rows // blk,),
        in_specs=[row, pl.BlockSpec((blk, 1), lambda i: (i, 0)),
                  full, full, full, full, full, vec],
        out_specs=[row, row, row, row],
        out_shape=[jax.ShapeDtypeStruct((rows, D), jnp.float32),
                   jax.ShapeDtypeStruct((rows, D), jnp.float32),
                   jax.ShapeDtypeStruct((rows, D), jnp.float32),
                   jax.ShapeDtypeStruct((rows, D), jnp.float32)],
    )(x, deg, wtx, wtd, w0, w1, wx, bias)


def _relu_half(h):
    col = lax.broadcasted_iota(jnp.int32, h.shape, 1)
    return jnp.where(col >= D // 2, jnp.maximum(h, 0.0), h)


def _tc_stats(parts, *, blk):
    rows = parts[0].shape[0]
    n = len(parts)

    def body(*refs):
        in_refs, h_ref, stats_ref = refs[:n], refs[n], refs[n + 1]
        h = in_refs[0][...].astype(jnp.float32)
        for r in in_refs[1:]:
            h = h + r[...].astype(jnp.float32)
        h = _relu_half(h)
        h_ref[...] = h

        @pl.when(pl.program_id(0) == 0)
        def _():
            stats_ref[...] = jnp.zeros_like(stats_ref)

        stats_ref[0:1, :] = stats_ref[0:1, :] + jnp.sum(h, 0, keepdims=True)
        stats_ref[1:2, :] = stats_ref[1:2, :] + jnp.sum(h * h, 0,
                                                        keepdims=True)

    row = pl.BlockSpec((blk, D), lambda i: (i, 0))
    return pl.pallas_call(
        body,
        grid=(rows // blk,),
        in_specs=[row] * n,
        out_specs=[row, pl.BlockSpec((8, D), lambda i: (0, 0))],
        out_shape=[jax.ShapeDtypeStruct((rows, D), jnp.float32),
                   jax.ShapeDtypeStruct((8, D), jnp.float32)],
    )(*parts)


def _tc_bn(h, stats, s, b, *, blk):
    rows = h.shape[0]
    inv_n = 1.0 / rows

    def body(h_ref, stats_ref, s_ref, b_ref, o_ref):
        m = stats_ref[0:1, :] * inv_n
        v = stats_ref[1:2, :] * inv_n - m * m
        scale = lax.rsqrt(v + 1e-5) * s_ref[...]
        o_ref[...] = (h_ref[...] - m) * scale + b_ref[...]

    row = pl.BlockSpec((blk, D), lambda i: (i, 0))
    vec = pl.BlockSpec((1, D), lambda i: (0, 0))
    return pl.pallas_call(
        body,
        grid=(rows // blk,),
        in_specs=[row, pl.BlockSpec((8, D), lambda i: (0, 0)), vec, vec],
        out_specs=row,
        out_shape=jax.ShapeDtypeStruct((rows, D), jnp.float32),
    )(h, stats, s.reshape(1, D), b.reshape(1, D))


def kernel(x, y, deg_g, deg_lg, pm_pd, g_t, g_tt, lg_t, lg_tt, edge_dst,
           W_tx, b_tx, W_td, b_td, W_ty, b_ty, W_t0, b_t0, W_t1, b_t1,
           W_gy, b_gy, W_gd, b_gd, W_gx, b_gx, W_g0, b_g0, W_g1, b_g1,
           bnx_s, bnx_b, bny_s, bny_b):
    bias_x = (b_tx + b_td + b_t0 + b_t1 + b_ty).reshape(1, D)
    bias_y = (b_gy + b_gd + b_g0 + b_g1 + b_gx).reshape(1, D)

    ay, wy, z0p, z1p = _tc_pre_y(y, deg_lg, W_gy.T, W_gd.T, W_ty.T,
                                 W_g0.T, W_g1.T, bias_y, blk=2000)
    ax, u0p, u1p, zxp = _tc_pre_x(x, deg_g, W_tx.T, W_td.T,
                                  W_t0.T, W_t1.T, W_gx.T, bias_x, blk=2000)

    sx = _sc_gather_fused(u0p, u1p, g_t.reshape(-1), g_tt.reshape(-1),
                          zxp, pm_pd, rows=N, with_p=False)
    sy = _sc_gather_e(z0p, z1p, lg_t.reshape(-1), lg_tt.reshape(-1),
                      zxp, pm_pd)
    py = _sc_scatter_add(wy, edge_dst)

    hx, stx = _tc_stats([ax, sx, py[0], py[1]], blk=2000)
    hy, sty = _tc_stats([ay, sy], blk=2000)
    xn = _tc_bn(hx, stx, bnx_s, bnx_b, blk=2000)
    yn = _tc_bn(hy, sty, bny_s, bny_b, blk=2000)
    return (xn, yn)


# speedup vs baseline: 3.5186x; 1.4814x over previous
import functools

import jax
import jax.numpy as jnp
from jax import lax
from jax.experimental import pallas as pl
from jax.experimental.pallas import tpu as pltpu
from jax.experimental.pallas import tpu_sc as plsc

N = 10000
E = 160000
D = 128
K = 16
DW = D // 2

_info = plsc.get_sparse_core_info()
_NC = _info.num_cores
_NS = _info.num_subcores
_NW = _NC * _NS

_CH = 8


def _mesh():
    return plsc.VectorSubcoreMesh(core_axis_name="c", subcore_axis_name="s")


def _split_word(w):
    a = lax.bitcast_convert_type(lax.shift_left(w, 16), jnp.float32)
    b = lax.bitcast_convert_type(w, jnp.float32)
    return a, b


def _acc4_pair(load, n):
    aa, bb = [None] * 4, [None] * 4
    for kk in range(n):
        a, b = load(kk)
        j = kk % 4
        aa[j] = a if kk < 4 else aa[j] + a
        bb[j] = b if kk < 4 else bb[j] + b
    return (aa[0] + aa[1]) + (aa[2] + aa[3]), (bb[0] + bb[1]) + (bb[2] + bb[3])


@functools.partial(jax.jit, static_argnames=("rows", "with_p"))
def _sc_gather_fused(tabA, tabB, idxA, idxB, ptable, pidx, *, rows, with_p):
    nch = rows // _CH
    q, rem = divmod(nch, _NW)
    scratch = [
        pltpu.VMEM((_CH * K,), jnp.int32),
        pltpu.VMEM((_CH * K, DW), jnp.int32),
        pltpu.VMEM((_CH * K,), jnp.int32),
        pltpu.VMEM((_CH * K, DW), jnp.int32),
        pltpu.VMEM((_CH, D), jnp.float32),
        pltpu.SemaphoreType.DMA,
        pltpu.SemaphoreType.DMA,
        pltpu.SemaphoreType.DMA,
    ]
    if with_p:
        scratch += [
            pltpu.VMEM((_CH,), jnp.int32),
            pltpu.VMEM((_CH, D), jnp.float32),
            pltpu.SemaphoreType.DMA,
        ]

    def k(tabA_ref, tabB_ref, idxA_ref, idxB_ref, *rest):
        if with_p:
            (ptab_ref, pidx_ref, out,
             iA_v, rA_v, iB_v, rB_v, o_v, semA, semB, semO,
             iP_v, rP_v, semP) = rest
        else:
            (out, iA_v, rA_v, iB_v, rB_v, o_v, semA, semB, semO) = rest
        wid = lax.axis_index("s") * _NC + lax.axis_index("c")
        cnt = q + jnp.where(wid < rem, 1, 0)

        def start(ch, tab_ref, idx_ref, idx_v, rows_v, sem):
            pltpu.sync_copy(idx_ref.at[pl.ds(ch * _CH * K, _CH * K)], idx_v)
            pltpu.async_copy(tab_ref.at[idx_v], rows_v, sem)

        def startp(ch):
            pltpu.sync_copy(pidx_ref.at[pl.ds(ch * _CH, _CH)], iP_v)
            pltpu.async_copy(ptab_ref.at[iP_v], rP_v, semP)

        @pl.when(cnt > 0)
        def _():
            start(wid, tabA_ref, idxA_ref, iA_v, rA_v, semA)
            start(wid, tabB_ref, idxB_ref, iB_v, rB_v, semB)
            if with_p:
                startp(wid)

        def reduce_list(rows_v, first):
            @pl.loop(0, _CH)
            def _(r):
                for c in range(4):
                    sl = pl.ds(c * 16, 16)
                    sh = pl.ds(64 + c * 16, 16)
                    lo, hi = _acc4_pair(
                        lambda kk: _split_word(rows_v[r * K + kk, sl]), K)
                    if first:
                        o_v[r, sl] = lo
                        o_v[r, sh] = hi
                    else:
                        o_v[r, sl] = o_v[r, sl] + lo
                        o_v[r, sh] = o_v[r, sh] + hi

        @pl.loop(0, cnt)
        def _(i):
            ch = wid + i * _NW
            nxt = ch + _NW

            pltpu.make_async_copy(tabA_ref.at[iA_v], rA_v, semA).wait()

            @pl.when(i > 0)
            def _():
                pltpu.make_async_copy(
                    o_v, out.at[pl.ds((ch - _NW) * _CH, _CH)], semO).wait()

            reduce_list(rA_v, True)

            @pl.when(i + 1 < cnt)
            def _():
                start(nxt, tabA_ref, idxA_ref, iA_v, rA_v, semA)

            pltpu.make_async_copy(tabB_ref.at[iB_v], rB_v, semB).wait()
            reduce_list(rB_v, False)

            @pl.when(i + 1 < cnt)
            def _():
                start(nxt, tabB_ref, idxB_ref, iB_v, rB_v, semB)

            if with_p:
                pltpu.make_async_copy(ptab_ref.at[iP_v], rP_v, semP).wait()

                @pl.loop(0, _CH)
                def _(r):
                    for c in range(8):
                        sl = pl.ds(c * 16, 16)
                        o_v[r, sl] = o_v[r, sl] + rP_v[r, sl]

                @pl.when(i + 1 < cnt)
                def _():
                    startp(nxt)

            pltpu.async_copy(o_v, out.at[pl.ds(ch * _CH, _CH)], semO)

        @pl.when(cnt > 0)
        def _():
            last = wid + (cnt - 1) * _NW
            pltpu.make_async_copy(
                o_v, out.at[pl.ds(last * _CH, _CH)], semO).wait()

    built = pl.kernel(k,
                      out_type=jax.ShapeDtypeStruct((rows, D), jnp.float32),
                      mesh=_mesh(), scratch_types=scratch,
                      compiler_params=pltpu.CompilerParams(
                          use_tc_tiling_on_sc=False))
    if with_p:
        return built(tabA, tabB, idxA, idxB, ptable, pidx)
    return built(tabA, tabB, idxA, idxB)


_IB = 25


@jax.jit
def _sc_gather_e(tabA, tabB, idxA, idxB, ptable, pidx):
    cnt = E // _CH // _NW
    nb = cnt // _IB
    BR = _IB * _CH

    @functools.partial(
        pl.kernel,
        out_type=jax.ShapeDtypeStruct((E, D), jnp.float32),
        mesh=_mesh(),
        compiler_params=pltpu.CompilerParams(use_tc_tiling_on_sc=False),
        scratch_types=[
            pltpu.VMEM((_IB * _CH * K,), jnp.int32),
            pltpu.VMEM((_IB * _CH * K,), jnp.int32),
            pltpu.VMEM((BR,), jnp.int32),
            pltpu.VMEM((_CH * K, DW), jnp.int32),
            pltpu.VMEM((_CH * K, DW), jnp.int32),
            pltpu.VMEM((_CH * K, DW), jnp.int32),
            pltpu.VMEM((_CH * K, DW), jnp.int32),
            pltpu.VMEM((BR, DW), jnp.int32),
            pltpu.VMEM((BR, D), jnp.float32),
            pltpu.SemaphoreType.DMA,
            pltpu.SemaphoreType.DMA,
            pltpu.SemaphoreType.DMA,
            pltpu.SemaphoreType.DMA,
            pltpu.SemaphoreType.DMA,
            pltpu.SemaphoreType.DMA,
        ],
    )
    def k(tabA_ref, tabB_ref, idxA_ref, idxB_ref, ptab_ref, pidx_ref, out,
          iA_v, iB_v, iP_v, rA0_v, rA1_v, rB0_v, rB1_v, rP_v, o_v,
          semA0, semA1, semB0, semB1, semP, semO):
        wid = lax.axis_index("s") * _NC + lax.axis_index("c")
        s0 = wid * cnt

        def startA(t, buf, sem):
            pltpu.async_copy(tabA_ref.at[iA_v.at[pl.ds(t * _CH * K,
                                                       _CH * K)]],
                             buf, sem)

        def startB(t, buf, sem):
            pltpu.async_copy(tabB_ref.at[iB_v.at[pl.ds(t * _CH * K,
                                                       _CH * K)]],
                             buf, sem)

        def red_first(buf, t):
            @pl.loop(0, _CH)
            def _(r):
                for c in range(4):
                    sl = pl.ds(c * 16, 16)
                    sh = pl.ds(64 + c * 16, 16)
                    lo, hi = _acc4_pair(
                        lambda kk: _split_word(buf[r * K + kk, sl]), K)
                    o_v[t * _CH + r, sl] = lo
                    o_v[t * _CH + r, sh] = hi

        def red_add(buf, t):
            @pl.loop(0, _CH)
            def _(r):
                for c in range(4):
                    sl = pl.ds(c * 16, 16)
                    sh = pl.ds(64 + c * 16, 16)
                    lo, hi = _acc4_pair(
                        lambda kk: _split_word(buf[r * K + kk, sl]), K)
                    o_v[t * _CH + r, sl] = o_v[t * _CH + r, sl] + lo
                    o_v[t * _CH + r, sh] = o_v[t * _CH + r, sh] + hi

        def padd(t):
            @pl.loop(0, _CH)
            def _(r):
                for c in range(4):
                    sl = pl.ds(c * 16, 16)
                    sh = pl.ds(64 + c * 16, 16)
                    lo, hi = _split_word(rP_v[t * _CH + r, sl])
                    o_v[t * _CH + r, sl] = o_v[t * _CH + r, sl] + lo
                    o_v[t * _CH + r, sh] = o_v[t * _CH + r, sh] + hi

        def slot(t, rA_v, semA, rB_v, semB):
            pltpu.make_async_copy(
                tabA_ref.at[iA_v.at[pl.ds(t * _CH * K, _CH * K)]],
                rA_v, semA).wait()
            red_first(rA_v, t)

            @pl.when(t + 2 < _IB)
            def _():
                startA(t + 2, rA_v, semA)

            pltpu.make_async_copy(
                tabB_ref.at[iB_v.at[pl.ds(t * _CH * K, _CH * K)]],
                rB_v, semB).wait()
            red_add(rB_v, t)

            @pl.when(t + 2 < _IB)
            def _():
                startB(t + 2, rB_v, semB)

            padd(t)

        @pl.loop(0, nb)
        def _(b):
            bc = s0 + b * _IB

            @pl.when(b > 0)
            def _():
                pltpu.make_async_copy(
                    o_v, out.at[pl.ds((bc - _IB) * _CH, BR)], semO).wait()

            pltpu.sync_copy(idxA_ref.at[pl.ds(bc * _CH * K, _IB * _CH * K)],
                            iA_v)
            pltpu.sync_copy(idxB_ref.at[pl.ds(bc * _CH * K, _IB * _CH * K)],
                            iB_v)
            pltpu.sync_copy(pidx_ref.at[pl.ds(bc * _CH, BR)], iP_v)
            startA(0, rA0_v, semA0)
            startA(1, rA1_v, semA1)
            startB(0, rB0_v, semB0)
            startB(1, rB1_v, semB1)
            pltpu.async_copy(ptab_ref.at[iP_v.at[pl.ds(0, 128)]],
                             rP_v.at[pl.ds(0, 128)], semP)
            pltpu.async_copy(ptab_ref.at[iP_v.at[pl.ds(128, BR - 128)]],
                             rP_v.at[pl.ds(128, BR - 128)], semP)
            pltpu.make_async_copy(
                ptab_ref.at[iP_v.at[pl.ds(128, BR - 128)]],
                rP_v.at[pl.ds(128, BR - 128)], semP).wait()
            pltpu.make_async_copy(
                ptab_ref.at[iP_v.at[pl.ds(0, 128)]],
                rP_v.at[pl.ds(0, 128)], semP).wait()

            @pl.loop(0, _IB, step=2)
            def _(t):
                slot(t, rA0_v, semA0, rB0_v, semB0)

                @pl.when(t + 1 < _IB)
                def _():
                    slot(t + 1, rA1_v, semA1, rB1_v, semB1)

            pltpu.async_copy(o_v, out.at[pl.ds(bc * _CH, BR)], semO)

        pltpu.make_async_copy(
            o_v, out.at[pl.ds((s0 + (nb - 1) * _IB) * _CH, BR)], semO).wait()

    return k(tabA, tabB, idxA, idxB, ptable, pidx)


@jax.jit
def _sc_scatter_add(vals, dst):
    CH = 64
    nch = E // CH
    q, rem = divmod(nch, _NW)
    RB = 16
    nrch = N // RB
    rq, rrem = divmod(nrch, _NS)

    @functools.partial(
        pl.kernel,
        out_type=jax.ShapeDtypeStruct((_NC, N, D), jnp.float32),
        mesh=_mesh(),
        scratch_types=[
            pltpu.VMEM((CH,), jnp.int32),
            pltpu.VMEM((CH, D), jnp.float32),
            pltpu.VMEM((CH,), jnp.int32),
            pltpu.VMEM((CH, D), jnp.float32),
            pltpu.VMEM((RB, D), jnp.float32),
            pltpu.VMEM((RB, D), jnp.float32),
            pltpu.VMEM_SHARED((N, D), jnp.float32),
            pltpu.SemaphoreType.DMA,
            pltpu.SemaphoreType.DMA,
            pltpu.SemaphoreType.DMA,
            pltpu.SemaphoreType.DMA,
            pltpu.SemaphoreType.DMA,
            pltpu.SemaphoreType.DMA,
        ],
    )
    def k(vals_ref, dst_ref, out_ref, idx0_v, rows0_v, idx1_v, rows1_v,
          zbuf, obuf, acc, semI0, semR0, semS0, semI1, semR1, semS1):
        cid = lax.axis_index("c")
        sid = lax.axis_index("s")
        wid = sid * _NC + cid
        rcnt = rq + jnp.where(sid < rrem, 1, 0)

        for r in range(RB):
            for c in range(D // 16):
                zbuf[r, pl.ds(c * 16, 16)] = jnp.zeros((16,), jnp.float32)

        @pl.loop(0, rcnt)
        def _(j):
            pltpu.sync_copy(zbuf, acc.at[pl.ds((sid + j * _NS) * RB, RB)])

        plsc.subcore_barrier()

        cnt = q + jnp.where(wid < rem, 1, 0)
        cnt0 = (cnt + 1) // 2
        cnt1 = cnt // 2
        streams = ((cnt0, 0, idx0_v, rows0_v, semI0, semR0, semS0),
                   (cnt1, 1, idx1_v, rows1_v, semI1, semR1, semS1))

        def load(s, idx_v, rows_v, semI, semR, j):
            ch = wid + (2 * j + s) * _NW
            pltpu.async_copy(dst_ref.at[pl.ds(ch * CH, CH)], idx_v, semI)
            pltpu.async_copy(vals_ref.at[pl.ds(ch * CH, CH)], rows_v, semR)

        for cs, s, idx_v, rows_v, semI, semR, semS in streams:
            @pl.when(cs > 0)
            def _():
                load(s, idx_v, rows_v, semI, semR, 0)

        @pl.loop(0, cnt0)
        def _(j):
            for cs, s, idx_v, rows_v, semI, semR, semS in streams:
                @pl.when(j < cs)
                def _():
                    ch = wid + (2 * j + s) * _NW
                    pltpu.make_async_copy(
                        dst_ref.at[pl.ds(ch * CH, CH)], idx_v, semI).wait()
                    pltpu.make_async_copy(
                        vals_ref.at[pl.ds(ch * CH, CH)], rows_v, semR).wait()
                    pltpu.async_copy(rows_v, acc.at[idx_v], semS, add=True)

            for cs, s, idx_v, rows_v, semI, semR, semS in streams:
                @pl.when(j < cs)
                def _():
                    pltpu.make_async_copy(rows_v, acc.at[idx_v], semS).wait()

                    @pl.when(j + 1 < cs)
                    def _():
                        load(s, idx_v, rows_v, semI, semR, j + 1)

        plsc.subcore_barrier()

        @pl.loop(0, rcnt)
        def _(j):
            off = (sid + j * _NS) * RB
            pltpu.sync_copy(acc.at[pl.ds(off, RB)], obuf)
            pltpu.sync_copy(obuf, out_ref.at[cid, pl.ds(off, RB)])

    return k(vals, dst)


def _dot(a, b):
    return jnp.dot(a, b, preferred_element_type=jnp.float32)


def _pack_rows(z):
    t = lax.bitcast_convert_type(z, jnp.int32)
    rnd = jnp.bitwise_and(lax.shift_right_logical(t, 16), 1)
    bits = lax.shift_right_logical(t + 0x7FFF + rnd, 16)
    lo = bits[:, :DW]
    hi = bits[:, DW:]
    return jnp.bitwise_or(lo, lax.shift_left(hi, 16))


def _tc_pre_y(y, deg, wgy, wgd, wty, w0, w1, bias, *, blk):
    rows = y.shape[0]

    def body(y_ref, d_ref, wgy_r, wgd_r, wty_r, w0_r, w1_r, b_ref,
             ay_ref, wy_ref, z0_ref, z1_ref):
        yb = y_ref[...]
        ay_ref[...] = (_dot(yb, wgy_r[...]) + _dot(yb * d_ref[...], wgd_r[...])
                       + b_ref[...])
        wy_ref[...] = _dot(yb, wty_r[...])
        z0_ref[...] = _pack_rows(_dot(yb, w0_r[...]))
        z1_ref[...] = _pack_rows(_dot(yb, w1_r[...]))

    row = pl.BlockSpec((blk, D), lambda i: (i, 0))
    roww = pl.BlockSpec((blk, DW), lambda i: (i, 0))
    full = pl.BlockSpec((D, D), lambda i: (0, 0))
    vec = pl.BlockSpec((1, D), lambda i: (0, 0))
    return pl.pallas_call(
        body,
        grid=(rows // blk,),
        in_specs=[row, pl.BlockSpec((blk, 1), lambda i: (i, 0)),
                  full, full, full, full, full, vec],
        out_specs=[row, row, roww, roww],
        out_shape=[jax.ShapeDtypeStruct((rows, D), jnp.float32),
                   jax.ShapeDtypeStruct((rows, D), jnp.float32),
                   jax.ShapeDtypeStruct((rows, DW), jnp.int32),
                   jax.ShapeDtypeStruct((rows, DW), jnp.int32)],
    )(y, deg, wgy, wgd, wty, w0, w1, bias)


def _tc_pre_x(x, deg, wtx, wtd, w0, w1, wx, bias, *, blk):
    rows = x.shape[0]

    def body(x_ref, d_ref, wtx_r, wtd_r, w0_r, w1_r, wx_r, b_ref,
             ax_ref, u0_ref, u1_ref, zx_ref):
        xb = x_ref[...]
        ax_ref[...] = (_dot(xb, wtx_r[...]) + _dot(xb * d_ref[...], wtd_r[...])
                       + b_ref[...])
        u0_ref[...] = _pack_rows(_dot(xb, w0_r[...]))
        u1_ref[...] = _pack_rows(_dot(xb, w1_r[...]))
        zx_ref[...] = _pack_rows(_dot(xb, wx_r[...]))

    row = pl.BlockSpec((blk, D), lambda i: (i, 0))
    roww = pl.BlockSpec((blk, DW), lambda i: (i, 0))
    full = pl.BlockSpec((D, D), lambda i: (0, 0))
    vec = pl.BlockSpec((1, D), lambda i: (0, 0))
    return pl.pallas_call(
        body,
        grid=(rows // blk,),
        in_specs=[row, pl.BlockSpec((blk, 1), lambda i: (i, 0)),
                  full, full, full, full, full, vec],
        out_specs=[row, roww, roww, roww],
        out_shape=[jax.ShapeDtypeStruct((rows, D), jnp.float32),
                   jax.ShapeDtypeStruct((rows, DW), jnp.int32),
                   jax.ShapeDtypeStruct((rows, DW), jnp.int32),
                   jax.ShapeDtypeStruct((rows, DW), jnp.int32)],
    )(x, deg, wtx, wtd, w0, w1, wx, bias)


def _relu_half(h):
    col = lax.broadcasted_iota(jnp.int32, h.shape, 1)
    return jnp.where(col >= D // 2, jnp.maximum(h, 0.0), h)


def _tc_stats(parts, *, blk):
    rows = parts[0].shape[0]
    n = len(parts)

    def body(*refs):
        in_refs, h_ref, stats_ref = refs[:n], refs[n], refs[n + 1]
        h = in_refs[0][...].astype(jnp.float32)
        for r in in_refs[1:]:
            h = h + r[...].astype(jnp.float32)
        h = _relu_half(h)
        h_ref[...] = h

        @pl.when(pl.program_id(0) == 0)
        def _():
            stats_ref[...] = jnp.zeros_like(stats_ref)

        stats_ref[0:1, :] = stats_ref[0:1, :] + jnp.sum(h, 0, keepdims=True)
        stats_ref[1:2, :] = stats_ref[1:2, :] + jnp.sum(h * h, 0,
                                                        keepdims=True)

    row = pl.BlockSpec((blk, D), lambda i: (i, 0))
    return pl.pallas_call(
        body,
        grid=(rows // blk,),
        in_specs=[row] * n,
        out_specs=[row, pl.BlockSpec((8, D), lambda i: (0, 0))],
        out_shape=[jax.ShapeDtypeStruct((rows, D), jnp.float32),
                   jax.ShapeDtypeStruct((8, D), jnp.float32)],
    )(*parts)


def _tc_bn(h, stats, s, b, *, blk):
    rows = h.shape[0]
    inv_n = 1.0 / rows

    def body(h_ref, stats_ref, s_ref, b_ref, o_ref):
        m = stats_ref[0:1, :] * inv_n
        v = stats_ref[1:2, :] * inv_n - m * m
        scale = lax.rsqrt(v + 1e-5) * s_ref[...]
        o_ref[...] = (h_ref[...] - m) * scale + b_ref[...]

    row = pl.BlockSpec((blk, D), lambda i: (i, 0))
    vec = pl.BlockSpec((1, D), lambda i: (0, 0))
    return pl.pallas_call(
        body,
        grid=(rows // blk,),
        in_specs=[row, pl.BlockSpec((8, D), lambda i: (0, 0)), vec, vec],
        out_specs=row,
        out_shape=jax.ShapeDtypeStruct((rows, D), jnp.float32),
    )(h, stats, s.reshape(1, D), b.reshape(1, D))


def kernel(x, y, deg_g, deg_lg, pm_pd, g_t, g_tt, lg_t, lg_tt, edge_dst,
           W_tx, b_tx, W_td, b_td, W_ty, b_ty, W_t0, b_t0, W_t1, b_t1,
           W_gy, b_gy, W_gd, b_gd, W_gx, b_gx, W_g0, b_g0, W_g1, b_g1,
           bnx_s, bnx_b, bny_s, bny_b):
    bias_x = (b_tx + b_td + b_t0 + b_t1 + b_ty).reshape(1, D)
    bias_y = (b_gy + b_gd + b_g0 + b_g1 + b_gx).reshape(1, D)

    ay, wy, z0p, z1p = _tc_pre_y(y, deg_lg, W_gy.T, W_gd.T, W_ty.T,
                                 W_g0.T, W_g1.T, bias_y, blk=2000)
    ax, u0p, u1p, zxp = _tc_pre_x(x, deg_g, W_tx.T, W_td.T,
                                  W_t0.T, W_t1.T, W_gx.T, bias_x, blk=2000)

    sx = _sc_gather_fused(u0p, u1p, g_t.reshape(-1), g_tt.reshape(-1),
                          zxp, pm_pd, rows=N, with_p=False)
    py = _sc_scatter_add(wy, edge_dst)
    sy = _sc_gather_e(z0p, z1p, lg_t.reshape(-1), lg_tt.reshape(-1),
                      zxp, pm_pd)

    hx, stx = _tc_stats([ax, sx, py[0], py[1]], blk=2000)
    hy, sty = _tc_stats([ay, sy], blk=2000)
    xn = _tc_bn(hx, stx, bnx_s, bnx_b, blk=2000)
    yn = _tc_bn(hy, sty, bny_s, bny_b, blk=2000)
    return (xn, yn)
